# Initial kernel scaffold; baseline (speedup 1.0000x reference)
import jax
import jax.numpy as jnp
from jax.experimental import pallas as pl


def kernel(x, edge_index, W1, b1, W2, b2, W3, b3, Wc, bc):
    raise NotImplementedError("write your pallas kernel here")


# speedup vs baseline: 11.7935x; 11.7935x over previous
import functools

import jax
import jax.numpy as jnp
from jax import lax
from jax.experimental import pallas as pl
from jax.experimental.pallas import tpu as pltpu
from jax.experimental.pallas import tpu_sc as plsc

N = 10000
F = 128
H = 32
E = 320000

NTILES = 32
EPT = E // NTILES
BLK = 128
NB = -(-EPT // BLK)
EPAD = NB * BLK
NACC = 10112
STRIPE = NACC // 16
PADROW = N
DW = 16

TM = 400


def _sc_mesh():
    return plsc.VectorSubcoreMesh(core_axis_name="c", subcore_axis_name="s")


def _sc_degree(dst_idx, zeros_init, ones_rows):
    @functools.partial(
        pl.kernel,
        out_type=jax.ShapeDtypeStruct((2, NACC, DW), jnp.float32),
        mesh=_sc_mesh(),
        compiler_params=pltpu.CompilerParams(use_tc_tiling_on_sc=False),
        scratch_types=[
            pltpu.VMEM((NB, BLK), jnp.int32),
            pltpu.VMEM((BLK, DW), jnp.float32),
            pltpu.VMEM_SHARED((NACC, DW), jnp.float32),
        ],
    )
    def deg_kernel(dst_hbm, zeros_hbm, ones_hbm, out_hbm, idx_d, rows, acc):
        c = lax.axis_index("c")
        s = lax.axis_index("s")
        wid = c * 16 + s
        pltpu.sync_copy(zeros_hbm.at[pl.ds(s * STRIPE, STRIPE)],
                        acc.at[pl.ds(s * STRIPE, STRIPE)])
        pltpu.sync_copy(dst_hbm.at[wid], idx_d)
        pltpu.sync_copy(ones_hbm, rows)
        plsc.subcore_barrier()

        def body(j, carry):
            pltpu.sync_copy(rows, acc.at[idx_d.at[j]], add=True)
            return carry

        lax.fori_loop(0, NB, body, 0)
        plsc.subcore_barrier()
        pltpu.sync_copy(acc.at[pl.ds(s * STRIPE, STRIPE)],
                        out_hbm.at[c].at[pl.ds(s * STRIPE, STRIPE)])

    return deg_kernel(dst_idx, zeros_init, ones_rows)


def _sc_scatter(table, src_idx, dst_idx, zeros_init):
    @functools.partial(
        pl.kernel,
        out_type=jax.ShapeDtypeStruct((2, NACC, H), jnp.float32),
        mesh=_sc_mesh(),
        compiler_params=pltpu.CompilerParams(use_tc_tiling_on_sc=False),
        scratch_types=[
            pltpu.VMEM((NB, BLK), jnp.int32),
            pltpu.VMEM((NB, BLK), jnp.int32),
            pltpu.VMEM((BLK, H), jnp.float32),
            pltpu.VMEM_SHARED((NACC, H), jnp.float32),
            pltpu.SemaphoreType.DMA,
        ],
    )
    def scat_kernel(table_hbm, src_hbm, dst_hbm, zeros_hbm, out_hbm,
                    idx_s, idx_d, rows, acc, sem):
        c = lax.axis_index("c")
        s = lax.axis_index("s")
        wid = c * 16 + s
        pltpu.sync_copy(zeros_hbm.at[pl.ds(s * STRIPE, STRIPE)],
                        acc.at[pl.ds(s * STRIPE, STRIPE)])
        pltpu.sync_copy(src_hbm.at[wid], idx_s)
        pltpu.sync_copy(dst_hbm.at[wid], idx_d)
        plsc.subcore_barrier()

        def body(j, carry):
            pltpu.async_copy(table_hbm.at[idx_s.at[j]], rows, sem).wait()
            pltpu.sync_copy(rows, acc.at[idx_d.at[j]], add=True)
            return carry

        lax.fori_loop(0, NB, body, 0)
        plsc.subcore_barrier()
        pltpu.sync_copy(acc.at[pl.ds(s * STRIPE, STRIPE)],
                        out_hbm.at[c].at[pl.ds(s * STRIPE, STRIPE)])

    return scat_kernel(table, src_idx, dst_idx, zeros_init)


def _dinv_block(da_ref, db_ref):
    deg = da_ref[:, 0:1] + db_ref[:, 0:1] + 1.0
    return lax.rsqrt(deg)


def _tc_prep(x, W1, degA, degB):
    def body(x_ref, w_ref, da_ref, db_ref, out_ref):
        dinv = _dinv_block(da_ref, db_ref)
        h = jnp.dot(x_ref[...], w_ref[...], preferred_element_type=jnp.float32)
        out_ref[...] = h * dinv

    return pl.pallas_call(
        body,
        grid=(N // TM,),
        in_specs=[
            pl.BlockSpec((TM, F), lambda j: (j, 0)),
            pl.BlockSpec((F, H), lambda j: (0, 0)),
            pl.BlockSpec((TM, DW), lambda j: (j, 0)),
            pl.BlockSpec((TM, DW), lambda j: (j, 0)),
        ],
        out_specs=pl.BlockSpec((TM, H), lambda j: (j, 0)),
        out_shape=jax.ShapeDtypeStruct((N, H), jnp.float32),
    )(x, W1, degA, degB)


def _tc_mid(Sa, Sb, hp, degA, degB, b, Wn):
    def body(sa_ref, sb_ref, hp_ref, da_ref, db_ref, b_ref, w_ref, out_ref):
        dinv = _dinv_block(da_ref, db_ref)
        hcur = jnp.maximum(
            dinv * (sa_ref[...] + sb_ref[...] + hp_ref[...]) + b_ref[...], 0.0)
        out_ref[...] = jnp.dot(
            hcur, w_ref[...], preferred_element_type=jnp.float32) * dinv

    return pl.pallas_call(
        body,
        grid=(N // TM,),
        in_specs=[
            pl.BlockSpec((TM, H), lambda j: (j, 0)),
            pl.BlockSpec((TM, H), lambda j: (j, 0)),
            pl.BlockSpec((TM, H), lambda j: (j, 0)),
            pl.BlockSpec((TM, DW), lambda j: (j, 0)),
            pl.BlockSpec((TM, DW), lambda j: (j, 0)),
            pl.BlockSpec((1, H), lambda j: (0, 0)),
            pl.BlockSpec((H, H), lambda j: (0, 0)),
        ],
        out_specs=pl.BlockSpec((TM, H), lambda j: (j, 0)),
        out_shape=jax.ShapeDtypeStruct((N, H), jnp.float32),
    )(Sa, Sb, hp, degA, degB, b, Wn)


def _tc_final(Sa, Sb, hp, degA, degB, b):
    def body(sa_ref, sb_ref, hp_ref, da_ref, db_ref, b_ref, out_ref):
        dinv = _dinv_block(da_ref, db_ref)
        out_ref[...] = jnp.maximum(
            dinv * (sa_ref[...] + sb_ref[...] + hp_ref[...]) + b_ref[...], 0.0)

    return pl.pallas_call(
        body,
        grid=(N // TM,),
        in_specs=[
            pl.BlockSpec((TM, H), lambda j: (j, 0)),
            pl.BlockSpec((TM, H), lambda j: (j, 0)),
            pl.BlockSpec((TM, H), lambda j: (j, 0)),
            pl.BlockSpec((TM, DW), lambda j: (j, 0)),
            pl.BlockSpec((TM, DW), lambda j: (j, 0)),
            pl.BlockSpec((1, H), lambda j: (0, 0)),
        ],
        out_specs=pl.BlockSpec((TM, H), lambda j: (j, 0)),
        out_shape=jax.ShapeDtypeStruct((N, H), jnp.float32),
    )(Sa, Sb, hp, degA, degB, b)


def _tc_head(z, Wc, bc):
    def body(zb_ref, zfull_ref, wc_ref, bc_ref, out_ref, sv_ref, acc_ref):
        j = pl.program_id(0)
        zb = zb_ref[...]
        s = lax.dot_general(zb, zfull_ref[...], (((1,), (1,)), ((), ())),
                            preferred_element_type=jnp.float32)
        out_ref[...] = 1.0 / (1.0 + jnp.exp(-s))

        @pl.when(j == 0)
        def _():
            acc_ref[...] = jnp.zeros_like(acc_ref)

        acc_ref[...] += jnp.sum(zb, axis=0, keepdims=True)
        sv_ref[...] = (jnp.dot(acc_ref[...], wc_ref[...],
                               preferred_element_type=jnp.float32)
                       * (1.0 / N)) + bc_ref[...]

    return pl.pallas_call(
        body,
        grid=(N // TM,),
        in_specs=[
            pl.BlockSpec((TM, H), lambda j: (j, 0)),
            pl.BlockSpec((N, H), lambda j: (0, 0)),
            pl.BlockSpec((H, 1), lambda j: (0, 0)),
            pl.BlockSpec((1, 1), lambda j: (0, 0)),
        ],
        out_specs=[
            pl.BlockSpec((TM, N), lambda j: (j, 0)),
            pl.BlockSpec((1, 1), lambda j: (0, 0)),
        ],
        out_shape=[
            jax.ShapeDtypeStruct((N, N), jnp.float32),
            jax.ShapeDtypeStruct((1, 1), jnp.float32),
        ],
        scratch_shapes=[pltpu.VMEM((1, H), jnp.float32)],
    )(z, z, Wc, bc)


def kernel(x, edge_index, W1, b1, W2, b2, W3, b3, Wc, bc):
    src = edge_index[0].astype(jnp.int32)
    dst = edge_index[1].astype(jnp.int32)
    srcr = jnp.pad(src.reshape(NTILES, EPT), ((0, 0), (0, EPAD - EPT)),
                   constant_values=0).reshape(NTILES, NB, BLK)
    dstr = jnp.pad(dst.reshape(NTILES, EPT), ((0, 0), (0, EPAD - EPT)),
                   constant_values=PADROW).reshape(NTILES, NB, BLK)
    zeros_h = jnp.zeros((NACC, H), jnp.float32)
    zeros_d = jnp.zeros((NACC, DW), jnp.float32)
    ones_rows = jnp.ones((BLK, DW), jnp.float32)

    deg = _sc_degree(dstr, zeros_d, ones_rows)
    degA, degB = deg[0], deg[1]

    h1p = _tc_prep(x, W1, degA, degB)
    S1 = _sc_scatter(h1p, srcr, dstr, zeros_h)
    h2p = _tc_mid(S1[0], S1[1], h1p, degA, degB, b1.reshape(1, H), W2)
    S2 = _sc_scatter(h2p, srcr, dstr, zeros_h)
    h3p = _tc_mid(S2[0], S2[1], h2p, degA, degB, b2.reshape(1, H), W3)
    S3 = _sc_scatter(h3p, srcr, dstr, zeros_h)
    z = _tc_final(S3[0], S3[1], h3p, degA, degB, b3.reshape(1, H))

    scores, sv = _tc_head(z, Wc, bc.reshape(1, 1))
    return scores.reshape(-1), sv[0, 0]


# speedup vs baseline: 12.2032x; 1.0347x over previous
import functools

import jax
import jax.numpy as jnp
from jax import lax
from jax.experimental import pallas as pl
from jax.experimental.pallas import tpu as pltpu
from jax.experimental.pallas import tpu_sc as plsc

N = 10000
F = 128
H = 32
E = 320000

NTILES = 32
EPT = E // NTILES
BLK = 128
NBUF = 8
NB = 80
EPAD = NB * BLK
NACC = 10112
STRIPE = NACC // 16
PADROW = N
DW = 16

TM = 400


def _sc_mesh():
    return plsc.VectorSubcoreMesh(core_axis_name="c", subcore_axis_name="s")


def _sc_degree(dst_idx, zeros_init, ones_rows):
    @functools.partial(
        pl.kernel,
        out_type=jax.ShapeDtypeStruct((2, NACC, DW), jnp.float32),
        mesh=_sc_mesh(),
        compiler_params=pltpu.CompilerParams(use_tc_tiling_on_sc=False),
        scratch_types=[
            pltpu.VMEM((NB, BLK), jnp.int32),
            pltpu.VMEM((BLK, DW), jnp.float32),
            pltpu.VMEM_SHARED((NACC, DW), jnp.float32),
        ],
    )
    def deg_kernel(dst_hbm, zeros_hbm, ones_hbm, out_hbm, idx_d, rows, acc):
        c = lax.axis_index("c")
        s = lax.axis_index("s")
        wid = c * 16 + s
        pltpu.sync_copy(zeros_hbm.at[pl.ds(s * STRIPE, STRIPE)],
                        acc.at[pl.ds(s * STRIPE, STRIPE)])
        pltpu.sync_copy(dst_hbm.at[wid], idx_d)
        pltpu.sync_copy(ones_hbm, rows)
        plsc.subcore_barrier()

        def body(j, carry):
            pltpu.sync_copy(rows, acc.at[idx_d.at[j]], add=True)
            return carry

        lax.fori_loop(0, NB, body, 0)
        plsc.subcore_barrier()
        pltpu.sync_copy(acc.at[pl.ds(s * STRIPE, STRIPE)],
                        out_hbm.at[c].at[pl.ds(s * STRIPE, STRIPE)])

    return deg_kernel(dst_idx, zeros_init, ones_rows)


def _sc_scatter(table, src_idx, dst_idx, zeros_init):
    @functools.partial(
        pl.kernel,
        out_type=jax.ShapeDtypeStruct((2, NACC, H), jnp.float32),
        mesh=_sc_mesh(),
        compiler_params=pltpu.CompilerParams(use_tc_tiling_on_sc=False),
        scratch_types=[
            pltpu.VMEM((NB, BLK), jnp.int32),
            pltpu.VMEM((NB, BLK), jnp.int32),
            pltpu.VMEM((NBUF, BLK, H), jnp.float32),
            pltpu.VMEM_SHARED((NACC, H), jnp.float32),
            pltpu.SemaphoreType.DMA((NBUF,)),
            pltpu.SemaphoreType.DMA((NBUF,)),
        ],
    )
    def scat_kernel(table_hbm, src_hbm, dst_hbm, zeros_hbm, out_hbm,
                    idx_s, idx_d, rows, acc, gsem, ssem):
        c = lax.axis_index("c")
        s = lax.axis_index("s")
        wid = c * 16 + s
        pltpu.sync_copy(zeros_hbm.at[pl.ds(s * STRIPE, STRIPE)],
                        acc.at[pl.ds(s * STRIPE, STRIPE)])
        pltpu.sync_copy(src_hbm.at[wid], idx_s)
        pltpu.sync_copy(dst_hbm.at[wid], idx_d)
        plsc.subcore_barrier()

        def gather(j, b):
            return pltpu.make_async_copy(
                table_hbm.at[idx_s.at[j]], rows.at[b], gsem.at[b])

        def scatter(j, b):
            return pltpu.make_async_copy(
                rows.at[b], acc.at[idx_d.at[j]], ssem.at[b])

        for b in range(NBUF):
            pltpu.async_copy(table_hbm.at[idx_s.at[b]], rows.at[b],
                             gsem.at[b])

        def body(t, carry):
            o = t * NBUF
            for b in range(NBUF):
                gather(o + b, b).wait()
                pltpu.async_copy(rows.at[b], acc.at[idx_d.at[o + b]],
                                 ssem.at[b], add=True)
            for b in range(NBUF):
                scatter(o + b, b).wait()
                pltpu.async_copy(table_hbm.at[idx_s.at[o + NBUF + b]],
                                 rows.at[b], gsem.at[b])
            return carry

        lax.fori_loop(0, NB // NBUF - 1, body, 0)

        o = NB - NBUF
        for b in range(NBUF):
            gather(o + b, b).wait()
            pltpu.async_copy(rows.at[b], acc.at[idx_d.at[o + b]],
                             ssem.at[b], add=True)
        for b in range(NBUF):
            scatter(o + b, b).wait()
        plsc.subcore_barrier()
        pltpu.sync_copy(acc.at[pl.ds(s * STRIPE, STRIPE)],
                        out_hbm.at[c].at[pl.ds(s * STRIPE, STRIPE)])

    return scat_kernel(table, src_idx, dst_idx, zeros_init)


def _dinv_block(da_ref, db_ref):
    deg = da_ref[:, 0:1] + db_ref[:, 0:1] + 1.0
    return lax.rsqrt(deg)


def _tc_prep(x, W1, degA, degB):
    def body(x_ref, w_ref, da_ref, db_ref, out_ref):
        dinv = _dinv_block(da_ref, db_ref)
        h = jnp.dot(x_ref[...], w_ref[...], preferred_element_type=jnp.float32)
        out_ref[...] = h * dinv

    return pl.pallas_call(
        body,
        grid=(N // TM,),
        in_specs=[
            pl.BlockSpec((TM, F), lambda j: (j, 0)),
            pl.BlockSpec((F, H), lambda j: (0, 0)),
            pl.BlockSpec((TM, DW), lambda j: (j, 0)),
            pl.BlockSpec((TM, DW), lambda j: (j, 0)),
        ],
        out_specs=pl.BlockSpec((TM, H), lambda j: (j, 0)),
        out_shape=jax.ShapeDtypeStruct((N, H), jnp.float32),
    )(x, W1, degA, degB)


def _tc_mid(Sa, Sb, hp, degA, degB, b, Wn):
    def body(sa_ref, sb_ref, hp_ref, da_ref, db_ref, b_ref, w_ref, out_ref):
        dinv = _dinv_block(da_ref, db_ref)
        hcur = jnp.maximum(
            dinv * (sa_ref[...] + sb_ref[...] + hp_ref[...]) + b_ref[...], 0.0)
        out_ref[...] = jnp.dot(
            hcur, w_ref[...], preferred_element_type=jnp.float32) * dinv

    return pl.pallas_call(
        body,
        grid=(N // TM,),
        in_specs=[
            pl.BlockSpec((TM, H), lambda j: (j, 0)),
            pl.BlockSpec((TM, H), lambda j: (j, 0)),
            pl.BlockSpec((TM, H), lambda j: (j, 0)),
            pl.BlockSpec((TM, DW), lambda j: (j, 0)),
            pl.BlockSpec((TM, DW), lambda j: (j, 0)),
            pl.BlockSpec((1, H), lambda j: (0, 0)),
            pl.BlockSpec((H, H), lambda j: (0, 0)),
        ],
        out_specs=pl.BlockSpec((TM, H), lambda j: (j, 0)),
        out_shape=jax.ShapeDtypeStruct((N, H), jnp.float32),
    )(Sa, Sb, hp, degA, degB, b, Wn)


def _tc_final(Sa, Sb, hp, degA, degB, b):
    def body(sa_ref, sb_ref, hp_ref, da_ref, db_ref, b_ref, out_ref):
        dinv = _dinv_block(da_ref, db_ref)
        out_ref[...] = jnp.maximum(
            dinv * (sa_ref[...] + sb_ref[...] + hp_ref[...]) + b_ref[...], 0.0)

    return pl.pallas_call(
        body,
        grid=(N // TM,),
        in_specs=[
            pl.BlockSpec((TM, H), lambda j: (j, 0)),
            pl.BlockSpec((TM, H), lambda j: (j, 0)),
            pl.BlockSpec((TM, H), lambda j: (j, 0)),
            pl.BlockSpec((TM, DW), lambda j: (j, 0)),
            pl.BlockSpec((TM, DW), lambda j: (j, 0)),
            pl.BlockSpec((1, H), lambda j: (0, 0)),
        ],
        out_specs=pl.BlockSpec((TM, H), lambda j: (j, 0)),
        out_shape=jax.ShapeDtypeStruct((N, H), jnp.float32),
    )(Sa, Sb, hp, degA, degB, b)


def _tc_head(z, Wc, bc):
    def body(zb_ref, zfull_ref, wc_ref, bc_ref, out_ref, sv_ref, acc_ref):
        j = pl.program_id(0)
        zb = zb_ref[...]
        s = lax.dot_general(zb, zfull_ref[...], (((1,), (1,)), ((), ())),
                            preferred_element_type=jnp.float32)
        out_ref[...] = 1.0 / (1.0 + jnp.exp(-s))

        @pl.when(j == 0)
        def _():
            acc_ref[...] = jnp.zeros_like(acc_ref)

        acc_ref[...] += jnp.sum(zb, axis=0, keepdims=True)
        sv_ref[...] = (jnp.dot(acc_ref[...], wc_ref[...],
                               preferred_element_type=jnp.float32)
                       * (1.0 / N)) + bc_ref[...]

    return pl.pallas_call(
        body,
        grid=(N // TM,),
        in_specs=[
            pl.BlockSpec((TM, H), lambda j: (j, 0)),
            pl.BlockSpec((N, H), lambda j: (0, 0)),
            pl.BlockSpec((H, 1), lambda j: (0, 0)),
            pl.BlockSpec((1, 1), lambda j: (0, 0)),
        ],
        out_specs=[
            pl.BlockSpec((TM, N), lambda j: (j, 0)),
            pl.BlockSpec((1, 1), lambda j: (0, 0)),
        ],
        out_shape=[
            jax.ShapeDtypeStruct((N, N), jnp.float32),
            jax.ShapeDtypeStruct((1, 1), jnp.float32),
        ],
        scratch_shapes=[pltpu.VMEM((1, H), jnp.float32)],
    )(z, z, Wc, bc)


def kernel(x, edge_index, W1, b1, W2, b2, W3, b3, Wc, bc):
    src = edge_index[0].astype(jnp.int32)
    dst = edge_index[1].astype(jnp.int32)
    srcr = jnp.pad(src.reshape(NTILES, EPT), ((0, 0), (0, EPAD - EPT)),
                   constant_values=0).reshape(NTILES, NB, BLK)
    dstr = jnp.pad(dst.reshape(NTILES, EPT), ((0, 0), (0, EPAD - EPT)),
                   constant_values=PADROW).reshape(NTILES, NB, BLK)
    zeros_h = jnp.zeros((NACC, H), jnp.float32)
    zeros_d = jnp.zeros((NACC, DW), jnp.float32)
    ones_rows = jnp.ones((BLK, DW), jnp.float32)

    deg = _sc_degree(dstr, zeros_d, ones_rows)
    degA, degB = deg[0], deg[1]

    h1p = _tc_prep(x, W1, degA, degB)
    S1 = _sc_scatter(h1p, srcr, dstr, zeros_h)
    h2p = _tc_mid(S1[0], S1[1], h1p, degA, degB, b1.reshape(1, H), W2)
    S2 = _sc_scatter(h2p, srcr, dstr, zeros_h)
    h3p = _tc_mid(S2[0], S2[1], h2p, degA, degB, b2.reshape(1, H), W3)
    S3 = _sc_scatter(h3p, srcr, dstr, zeros_h)
    z = _tc_final(S3[0], S3[1], h3p, degA, degB, b3.reshape(1, H))

    scores, sv = _tc_head(z, Wc, bc.reshape(1, 1))
    return scores.reshape(-1), sv[0, 0]


# speedup vs baseline: 16.7078x; 1.3691x over previous
import functools

import jax
import jax.numpy as jnp
from jax import lax
from jax.experimental import pallas as pl
from jax.experimental.pallas import tpu as pltpu
from jax.experimental.pallas import tpu_sc as plsc

N = 10000
F = 128
H = 32
E = 320000

NTILES = 32
EPT = E // NTILES
BLK = 100
NB = EPT // BLK
NBUF = 10
NACC = 10112
STRIPE = NACC // 16
DW = 16

TM = 400


def _sc_mesh():
    return plsc.VectorSubcoreMesh(core_axis_name="c", subcore_axis_name="s")


def _sc_degree(dst_idx, zeros_init, ones_rows):
    @functools.partial(
        pl.kernel,
        out_type=[jax.ShapeDtypeStruct((NACC, DW), jnp.float32),
                  jax.ShapeDtypeStruct((NACC, DW), jnp.float32)],
        mesh=_sc_mesh(),
        compiler_params=pltpu.CompilerParams(use_tc_tiling_on_sc=False),
        scratch_types=[
            pltpu.VMEM((NB, BLK), jnp.int32),
            pltpu.VMEM((BLK, DW), jnp.float32),
            pltpu.VMEM_SHARED((NACC, DW), jnp.float32),
        ],
    )
    def deg_kernel(dst_hbm, zeros_hbm, ones_hbm, outA, outB, idx_d, rows, acc):
        c = lax.axis_index("c")
        s = lax.axis_index("s")
        wid = c * 16 + s
        stripe = pl.ds(s * STRIPE, STRIPE)
        pltpu.sync_copy(zeros_hbm.at[stripe], acc.at[stripe])
        pltpu.sync_copy(dst_hbm.at[wid], idx_d)
        pltpu.sync_copy(ones_hbm, rows)
        plsc.subcore_barrier()

        def body(j, carry):
            pltpu.sync_copy(rows, acc.at[idx_d.at[j]], add=True)
            return carry

        lax.fori_loop(0, NB, body, 0)
        plsc.subcore_barrier()

        @pl.when(c == 0)
        def _():
            pltpu.sync_copy(acc.at[stripe], outA.at[stripe])

        @pl.when(c == 1)
        def _():
            pltpu.sync_copy(acc.at[stripe], outB.at[stripe])

    return deg_kernel(dst_idx, zeros_init, ones_rows)


def _sc_scatter(table, src_idx, dst_idx, zeros_init):
    @functools.partial(
        pl.kernel,
        out_type=[jax.ShapeDtypeStruct((NACC, H), jnp.float32),
                  jax.ShapeDtypeStruct((NACC, H), jnp.float32)],
        mesh=_sc_mesh(),
        compiler_params=pltpu.CompilerParams(use_tc_tiling_on_sc=False),
        scratch_types=[
            pltpu.VMEM((NB, BLK), jnp.int32),
            pltpu.VMEM((NB, BLK), jnp.int32),
            pltpu.VMEM((NBUF, BLK, H), jnp.float32),
            pltpu.VMEM_SHARED((NACC, H), jnp.float32),
            pltpu.SemaphoreType.DMA((NBUF,)),
            pltpu.SemaphoreType.DMA((NBUF,)),
        ],
    )
    def scat_kernel(table_hbm, src_hbm, dst_hbm, zeros_hbm, outA, outB,
                    idx_s, idx_d, rows, acc, gsem, ssem):
        c = lax.axis_index("c")
        s = lax.axis_index("s")
        wid = c * 16 + s
        stripe = pl.ds(s * STRIPE, STRIPE)
        pltpu.sync_copy(zeros_hbm.at[stripe], acc.at[stripe])
        pltpu.sync_copy(src_hbm.at[wid], idx_s)
        pltpu.sync_copy(dst_hbm.at[wid], idx_d)
        plsc.subcore_barrier()

        def gather(j, b):
            return pltpu.make_async_copy(
                table_hbm.at[idx_s.at[j]], rows.at[b], gsem.at[b])

        def scatter(j, b):
            return pltpu.make_async_copy(
                rows.at[b], acc.at[idx_d.at[j]], ssem.at[b])

        for b in range(NBUF):
            pltpu.async_copy(table_hbm.at[idx_s.at[b]], rows.at[b],
                             gsem.at[b])

        def body(t, carry):
            o = t * NBUF
            for b in range(NBUF):
                gather(o + b, b).wait()
                pltpu.async_copy(rows.at[b], acc.at[idx_d.at[o + b]],
                                 ssem.at[b], add=True)
            for b in range(NBUF):
                scatter(o + b, b).wait()
                pltpu.async_copy(table_hbm.at[idx_s.at[o + NBUF + b]],
                                 rows.at[b], gsem.at[b])
            return carry

        lax.fori_loop(0, NB // NBUF - 1, body, 0)

        o = NB - NBUF
        for b in range(NBUF):
            gather(o + b, b).wait()
            pltpu.async_copy(rows.at[b], acc.at[idx_d.at[o + b]],
                             ssem.at[b], add=True)
        for b in range(NBUF):
            scatter(o + b, b).wait()
        plsc.subcore_barrier()

        @pl.when(c == 0)
        def _():
            pltpu.sync_copy(acc.at[stripe], outA.at[stripe])

        @pl.when(c == 1)
        def _():
            pltpu.sync_copy(acc.at[stripe], outB.at[stripe])

    return scat_kernel(table, src_idx, dst_idx, zeros_init)


def _tc_prep(x, W1, degA, degB):
    def body(x_ref, w_ref, da_ref, db_ref, out_ref, dinv_ref):
        deg = da_ref[:, 0:1] + db_ref[:, 0:1] + 1.0
        dinv = lax.rsqrt(deg)
        dinv_ref[...] = dinv
        h = jnp.dot(x_ref[...], w_ref[...], preferred_element_type=jnp.float32)
        out_ref[...] = h * dinv

    return pl.pallas_call(
        body,
        grid=(1,),
        in_specs=[
            pl.BlockSpec((N, F), lambda j: (0, 0)),
            pl.BlockSpec((F, H), lambda j: (0, 0)),
            pl.BlockSpec((N, DW), lambda j: (0, 0)),
            pl.BlockSpec((N, DW), lambda j: (0, 0)),
        ],
        out_specs=[
            pl.BlockSpec((N, H), lambda j: (0, 0)),
            pl.BlockSpec((N, 1), lambda j: (0, 0)),
        ],
        out_shape=[
            jax.ShapeDtypeStruct((N, H), jnp.float32),
            jax.ShapeDtypeStruct((N, 1), jnp.float32),
        ],
    )(x, W1, degA, degB)


def _tc_mid(Sa, Sb, hp, dinv, b, Wn):
    def body(sa_ref, sb_ref, hp_ref, dinv_ref, b_ref, w_ref, out_ref):
        dinv = dinv_ref[...]
        hcur = jnp.maximum(
            dinv * (sa_ref[...] + sb_ref[...] + hp_ref[...]) + b_ref[...], 0.0)
        out_ref[...] = jnp.dot(
            hcur, w_ref[...], preferred_element_type=jnp.float32) * dinv

    return pl.pallas_call(
        body,
        grid=(1,),
        in_specs=[
            pl.BlockSpec((N, H), lambda j: (0, 0)),
            pl.BlockSpec((N, H), lambda j: (0, 0)),
            pl.BlockSpec((N, H), lambda j: (0, 0)),
            pl.BlockSpec((N, 1), lambda j: (0, 0)),
            pl.BlockSpec((1, H), lambda j: (0, 0)),
            pl.BlockSpec((H, H), lambda j: (0, 0)),
        ],
        out_specs=pl.BlockSpec((N, H), lambda j: (0, 0)),
        out_shape=jax.ShapeDtypeStruct((N, H), jnp.float32),
    )(Sa, Sb, hp, dinv, b, Wn)


def _tc_final(Sa, Sb, hp, dinv, b):
    def body(sa_ref, sb_ref, hp_ref, dinv_ref, b_ref, out_ref):
        out_ref[...] = jnp.maximum(
            dinv_ref[...] * (sa_ref[...] + sb_ref[...] + hp_ref[...])
            + b_ref[...], 0.0)

    return pl.pallas_call(
        body,
        grid=(1,),
        in_specs=[
            pl.BlockSpec((N, H), lambda j: (0, 0)),
            pl.BlockSpec((N, H), lambda j: (0, 0)),
            pl.BlockSpec((N, H), lambda j: (0, 0)),
            pl.BlockSpec((N, 1), lambda j: (0, 0)),
            pl.BlockSpec((1, H), lambda j: (0, 0)),
        ],
        out_specs=pl.BlockSpec((N, H), lambda j: (0, 0)),
        out_shape=jax.ShapeDtypeStruct((N, H), jnp.float32),
    )(Sa, Sb, hp, dinv, b)


def _tc_head(z, Wc, bc):
    def body(zb_ref, zfull_ref, wc_ref, bc_ref, out_ref, sv_ref, acc_ref):
        j = pl.program_id(0)
        zb = zb_ref[...]
        s = lax.dot_general(zb, zfull_ref[...], (((1,), (1,)), ((), ())),
                            preferred_element_type=jnp.float32)
        out_ref[...] = 1.0 / (1.0 + jnp.exp(-s))

        @pl.when(j == 0)
        def _():
            acc_ref[...] = jnp.zeros_like(acc_ref)

        acc_ref[...] += jnp.sum(zb, axis=0, keepdims=True)
        sv_ref[...] = (jnp.dot(acc_ref[...], wc_ref[...],
                               preferred_element_type=jnp.float32)
                       * (1.0 / N)) + bc_ref[...]

    return pl.pallas_call(
        body,
        grid=(N // TM,),
        in_specs=[
            pl.BlockSpec((TM, H), lambda j: (j, 0)),
            pl.BlockSpec((N, H), lambda j: (0, 0)),
            pl.BlockSpec((H, 1), lambda j: (0, 0)),
            pl.BlockSpec((1, 1), lambda j: (0, 0)),
        ],
        out_specs=[
            pl.BlockSpec((TM, N), lambda j: (j, 0)),
            pl.BlockSpec((1, 1), lambda j: (0, 0)),
        ],
        out_shape=[
            jax.ShapeDtypeStruct((N, N), jnp.float32),
            jax.ShapeDtypeStruct((1, 1), jnp.float32),
        ],
        scratch_shapes=[pltpu.VMEM((1, H), jnp.float32)],
    )(z, z, Wc, bc)


def kernel(x, edge_index, W1, b1, W2, b2, W3, b3, Wc, bc):
    src = edge_index[0].astype(jnp.int32)
    dst = edge_index[1].astype(jnp.int32)
    srcr = src.reshape(NTILES, NB, BLK)
    dstr = dst.reshape(NTILES, NB, BLK)
    zeros_h = jnp.zeros((NACC, H), jnp.float32)
    zeros_d = jnp.zeros((NACC, DW), jnp.float32)
    ones_rows = jnp.ones((BLK, DW), jnp.float32)

    degA, degB = _sc_degree(dstr, zeros_d, ones_rows)

    h1p, dinv = _tc_prep(x, W1, degA, degB)
    S1a, S1b = _sc_scatter(h1p, srcr, dstr, zeros_h)
    h2p = _tc_mid(S1a, S1b, h1p, dinv, b1.reshape(1, H), W2)
    S2a, S2b = _sc_scatter(h2p, srcr, dstr, zeros_h)
    h3p = _tc_mid(S2a, S2b, h2p, dinv, b2.reshape(1, H), W3)
    S3a, S3b = _sc_scatter(h3p, srcr, dstr, zeros_h)
    z = _tc_final(S3a, S3b, h3p, dinv, b3.reshape(1, H))

    scores, sv = _tc_head(z, Wc, bc.reshape(1, 1))
    return scores.reshape(-1), sv[0, 0]


# speedup vs baseline: 16.8444x; 1.0082x over previous
import functools

import jax
import jax.numpy as jnp
from jax import lax
from jax.experimental import pallas as pl
from jax.experimental.pallas import tpu as pltpu
from jax.experimental.pallas import tpu_sc as plsc

N = 10000
F = 128
H = 32
E = 320000

NTILES = 32
EPT = E // NTILES
BLK = 100
NB = EPT // BLK
NBUF = 10
NACC = 10112
STRIPE = NACC // 16
DW = 16

TM = 400


def _sc_mesh():
    return plsc.VectorSubcoreMesh(core_axis_name="c", subcore_axis_name="s")


def _sc_degree(edges, zeros_init, ones_rows):
    @functools.partial(
        pl.kernel,
        out_type=[jax.ShapeDtypeStruct((NACC, DW), jnp.float32),
                  jax.ShapeDtypeStruct((NACC, DW), jnp.float32)],
        mesh=_sc_mesh(),
        compiler_params=pltpu.CompilerParams(use_tc_tiling_on_sc=False),
        scratch_types=[
            pltpu.VMEM((NB, BLK), jnp.int32),
            pltpu.VMEM((BLK, DW), jnp.float32),
            pltpu.VMEM_SHARED((NACC, DW), jnp.float32),
        ],
    )
    def deg_kernel(edge_hbm, zeros_hbm, ones_hbm, outA, outB, idx_d, rows, acc):
        c = lax.axis_index("c")
        s = lax.axis_index("s")
        wid = c * 16 + s
        stripe = pl.ds(s * STRIPE, STRIPE)
        pltpu.sync_copy(zeros_hbm.at[stripe], acc.at[stripe])
        pltpu.sync_copy(edge_hbm.at[1, wid], idx_d)
        pltpu.sync_copy(ones_hbm, rows)
        plsc.subcore_barrier()

        def body(j, carry):
            pltpu.sync_copy(rows, acc.at[idx_d.at[j]], add=True)
            return carry

        lax.fori_loop(0, NB, body, 0)
        plsc.subcore_barrier()

        @pl.when(c == 0)
        def _():
            pltpu.sync_copy(acc.at[stripe], outA.at[stripe])

        @pl.when(c == 1)
        def _():
            pltpu.sync_copy(acc.at[stripe], outB.at[stripe])

    return deg_kernel(edges, zeros_init, ones_rows)


def _sc_scatter(table, edges, zeros_init):
    @functools.partial(
        pl.kernel,
        out_type=[jax.ShapeDtypeStruct((NACC, H), jnp.float32),
                  jax.ShapeDtypeStruct((NACC, H), jnp.float32)],
        mesh=_sc_mesh(),
        compiler_params=pltpu.CompilerParams(use_tc_tiling_on_sc=False),
        scratch_types=[
            pltpu.VMEM((NB, BLK), jnp.int32),
            pltpu.VMEM((NB, BLK), jnp.int32),
            pltpu.VMEM((NBUF, BLK, H), jnp.float32),
            pltpu.VMEM_SHARED((NACC, H), jnp.float32),
            pltpu.SemaphoreType.DMA((NBUF,)),
            pltpu.SemaphoreType.DMA((NBUF,)),
        ],
    )
    def scat_kernel(table_hbm, edge_hbm, zeros_hbm, outA, outB,
                    idx_s, idx_d, rows, acc, gsem, ssem):
        c = lax.axis_index("c")
        s = lax.axis_index("s")
        wid = c * 16 + s
        stripe = pl.ds(s * STRIPE, STRIPE)
        pltpu.sync_copy(zeros_hbm.at[stripe], acc.at[stripe])
        pltpu.sync_copy(edge_hbm.at[0, wid], idx_s)
        pltpu.sync_copy(edge_hbm.at[1, wid], idx_d)
        plsc.subcore_barrier()

        def gather(j, b):
            return pltpu.make_async_copy(
                table_hbm.at[idx_s.at[j]], rows.at[b], gsem.at[b])

        def scatter(j, b):
            return pltpu.make_async_copy(
                rows.at[b], acc.at[idx_d.at[j]], ssem.at[b])

        for b in range(NBUF):
            pltpu.async_copy(table_hbm.at[idx_s.at[b]], rows.at[b],
                             gsem.at[b])

        def body(t, carry):
            o = t * NBUF
            for b in range(NBUF):
                gather(o + b, b).wait()
                pltpu.async_copy(rows.at[b], acc.at[idx_d.at[o + b]],
                                 ssem.at[b], add=True)
            for b in range(NBUF):
                scatter(o + b, b).wait()
                pltpu.async_copy(table_hbm.at[idx_s.at[o + NBUF + b]],
                                 rows.at[b], gsem.at[b])
            return carry

        lax.fori_loop(0, NB // NBUF - 1, body, 0)

        o = NB - NBUF
        for b in range(NBUF):
            gather(o + b, b).wait()
            pltpu.async_copy(rows.at[b], acc.at[idx_d.at[o + b]],
                             ssem.at[b], add=True)
        for b in range(NBUF):
            scatter(o + b, b).wait()
        plsc.subcore_barrier()

        @pl.when(c == 0)
        def _():
            pltpu.sync_copy(acc.at[stripe], outA.at[stripe])

        @pl.when(c == 1)
        def _():
            pltpu.sync_copy(acc.at[stripe], outB.at[stripe])

    return scat_kernel(table, edges, zeros_init)


def _tc_prep(x, W1, degA, degB):
    def body(x_ref, w_ref, da_ref, db_ref, out_ref, dinv_ref):
        deg = da_ref[:, 0:1] + db_ref[:, 0:1] + 1.0
        dinv = lax.rsqrt(deg)
        dinv_ref[...] = dinv
        h = jnp.dot(x_ref[...], w_ref[...], preferred_element_type=jnp.float32)
        out_ref[...] = h * dinv

    return pl.pallas_call(
        body,
        grid=(1,),
        in_specs=[
            pl.BlockSpec((N, F), lambda j: (0, 0)),
            pl.BlockSpec((F, H), lambda j: (0, 0)),
            pl.BlockSpec((N, DW), lambda j: (0, 0)),
            pl.BlockSpec((N, DW), lambda j: (0, 0)),
        ],
        out_specs=[
            pl.BlockSpec((N, H), lambda j: (0, 0)),
            pl.BlockSpec((N, 1), lambda j: (0, 0)),
        ],
        out_shape=[
            jax.ShapeDtypeStruct((N, H), jnp.float32),
            jax.ShapeDtypeStruct((N, 1), jnp.float32),
        ],
    )(x, W1, degA, degB)


def _tc_mid(Sa, Sb, hp, dinv, b, Wn):
    def body(sa_ref, sb_ref, hp_ref, dinv_ref, b_ref, w_ref, out_ref):
        dinv = dinv_ref[...]
        hcur = jnp.maximum(
            dinv * (sa_ref[...] + sb_ref[...] + hp_ref[...]) + b_ref[...], 0.0)
        out_ref[...] = jnp.dot(
            hcur, w_ref[...], preferred_element_type=jnp.float32) * dinv

    return pl.pallas_call(
        body,
        grid=(1,),
        in_specs=[
            pl.BlockSpec((N, H), lambda j: (0, 0)),
            pl.BlockSpec((N, H), lambda j: (0, 0)),
            pl.BlockSpec((N, H), lambda j: (0, 0)),
            pl.BlockSpec((N, 1), lambda j: (0, 0)),
            pl.BlockSpec((1, H), lambda j: (0, 0)),
            pl.BlockSpec((H, H), lambda j: (0, 0)),
        ],
        out_specs=pl.BlockSpec((N, H), lambda j: (0, 0)),
        out_shape=jax.ShapeDtypeStruct((N, H), jnp.float32),
    )(Sa, Sb, hp, dinv, b, Wn)


def _tc_final(Sa, Sb, hp, dinv, b):
    def body(sa_ref, sb_ref, hp_ref, dinv_ref, b_ref, out_ref):
        out_ref[...] = jnp.maximum(
            dinv_ref[...] * (sa_ref[...] + sb_ref[...] + hp_ref[...])
            + b_ref[...], 0.0)

    return pl.pallas_call(
        body,
        grid=(1,),
        in_specs=[
            pl.BlockSpec((N, H), lambda j: (0, 0)),
            pl.BlockSpec((N, H), lambda j: (0, 0)),
            pl.BlockSpec((N, H), lambda j: (0, 0)),
            pl.BlockSpec((N, 1), lambda j: (0, 0)),
            pl.BlockSpec((1, H), lambda j: (0, 0)),
        ],
        out_specs=pl.BlockSpec((N, H), lambda j: (0, 0)),
        out_shape=jax.ShapeDtypeStruct((N, H), jnp.float32),
    )(Sa, Sb, hp, dinv, b)


def _tc_head(z, Wc, bc):
    def body(zb_ref, zfull_ref, wc_ref, bc_ref, out_ref, sv_ref, acc_ref):
        j = pl.program_id(0)
        zb = zb_ref[...]
        s = lax.dot_general(zb, zfull_ref[...], (((1,), (1,)), ((), ())),
                            preferred_element_type=jnp.float32)
        out_ref[...] = 1.0 / (1.0 + jnp.exp(-s))

        @pl.when(j == 0)
        def _():
            acc_ref[...] = jnp.zeros_like(acc_ref)

        acc_ref[...] += jnp.sum(zb, axis=0, keepdims=True)
        sv_ref[...] = (jnp.dot(acc_ref[...], wc_ref[...],
                               preferred_element_type=jnp.float32)
                       * (1.0 / N)) + bc_ref[...]

    return pl.pallas_call(
        body,
        grid=(N // TM,),
        in_specs=[
            pl.BlockSpec((TM, H), lambda j: (j, 0)),
            pl.BlockSpec((N, H), lambda j: (0, 0)),
            pl.BlockSpec((H, 1), lambda j: (0, 0)),
            pl.BlockSpec((1, 1), lambda j: (0, 0)),
        ],
        out_specs=[
            pl.BlockSpec((TM, N), lambda j: (j, 0)),
            pl.BlockSpec((1, 1), lambda j: (0, 0)),
        ],
        out_shape=[
            jax.ShapeDtypeStruct((N, N), jnp.float32),
            jax.ShapeDtypeStruct((1, 1), jnp.float32),
        ],
        scratch_shapes=[pltpu.VMEM((1, H), jnp.float32)],
    )(z, z, Wc, bc)


def kernel(x, edge_index, W1, b1, W2, b2, W3, b3, Wc, bc):
    edges = edge_index.astype(jnp.int32).reshape(2, NTILES, NB, BLK)
    zeros_h = jnp.zeros((NACC, H), jnp.float32)
    zeros_d = jnp.zeros((NACC, DW), jnp.float32)
    ones_rows = jnp.ones((BLK, DW), jnp.float32)

    degA, degB = _sc_degree(edges, zeros_d, ones_rows)

    h1p, dinv = _tc_prep(x, W1, degA, degB)
    S1a, S1b = _sc_scatter(h1p, edges, zeros_h)
    h2p = _tc_mid(S1a, S1b, h1p, dinv, b1.reshape(1, H), W2)
    S2a, S2b = _sc_scatter(h2p, edges, zeros_h)
    h3p = _tc_mid(S2a, S2b, h2p, dinv, b2.reshape(1, H), W3)
    S3a, S3b = _sc_scatter(h3p, edges, zeros_h)
    z = _tc_final(S3a, S3b, h3p, dinv, b3.reshape(1, H))

    scores, sv = _tc_head(z, Wc, bc.reshape(1, 1))
    return scores.reshape(-1), sv[0, 0]


# speedup vs baseline: 16.8783x; 1.0020x over previous
import functools

import jax
import jax.numpy as jnp
from jax import lax
from jax.experimental import pallas as pl
from jax.experimental.pallas import tpu as pltpu
from jax.experimental.pallas import tpu_sc as plsc

N = 10000
F = 128
H = 32
E = 320000

NTILES = 32
EPT = E // NTILES
BLK = 100
NB = EPT // BLK
NBUF = 10
NACC = 10112
STRIPE = NACC // 16
DW = 16

TM = 400


def _sc_mesh():
    return plsc.VectorSubcoreMesh(core_axis_name="c", subcore_axis_name="s")


def _sc_degree(edges, zeros_init, ones_rows):
    @functools.partial(
        pl.kernel,
        out_type=[jax.ShapeDtypeStruct((NACC, DW), jnp.float32),
                  jax.ShapeDtypeStruct((NACC, DW), jnp.float32)],
        mesh=_sc_mesh(),
        compiler_params=pltpu.CompilerParams(use_tc_tiling_on_sc=False),
        scratch_types=[
            pltpu.VMEM((NB, BLK), jnp.int32),
            pltpu.VMEM((BLK, DW), jnp.float32),
            pltpu.VMEM_SHARED((NACC, DW), jnp.float32),
        ],
    )
    def deg_kernel(edge_hbm, zeros_hbm, ones_hbm, outA, outB, idx_d, rows, acc):
        c = lax.axis_index("c")
        s = lax.axis_index("s")
        wid = c * 16 + s
        stripe = pl.ds(s * STRIPE, STRIPE)
        pltpu.sync_copy(zeros_hbm.at[stripe], acc.at[stripe])
        pltpu.sync_copy(edge_hbm.at[1, wid], idx_d)
        pltpu.sync_copy(ones_hbm, rows)
        plsc.subcore_barrier()

        def body(j, carry):
            pltpu.sync_copy(rows, acc.at[idx_d.at[j]], add=True)
            return carry

        lax.fori_loop(0, NB, body, 0)
        plsc.subcore_barrier()

        @pl.when(c == 0)
        def _():
            pltpu.sync_copy(acc.at[stripe], outA.at[stripe])

        @pl.when(c == 1)
        def _():
            pltpu.sync_copy(acc.at[stripe], outB.at[stripe])

    return deg_kernel(edges, zeros_init, ones_rows)


def _sc_scatter(table, edges, zeros_init):
    @functools.partial(
        pl.kernel,
        out_type=[jax.ShapeDtypeStruct((NACC, H), jnp.float32),
                  jax.ShapeDtypeStruct((NACC, H), jnp.float32)],
        mesh=_sc_mesh(),
        compiler_params=pltpu.CompilerParams(use_tc_tiling_on_sc=False),
        scratch_types=[
            pltpu.VMEM((NB, BLK), jnp.int32),
            pltpu.VMEM((NB, BLK), jnp.int32),
            pltpu.VMEM((NBUF, BLK, H), jnp.float32),
            pltpu.VMEM_SHARED((NACC, H), jnp.float32),
            pltpu.SemaphoreType.DMA((NBUF,)),
            pltpu.SemaphoreType.DMA((NBUF,)),
        ],
    )
    def scat_kernel(table_hbm, edge_hbm, zeros_hbm, outA, outB,
                    idx_s, idx_d, rows, acc, gsem, ssem):
        c = lax.axis_index("c")
        s = lax.axis_index("s")
        wid = c * 16 + s
        stripe = pl.ds(s * STRIPE, STRIPE)
        pltpu.sync_copy(zeros_hbm.at[stripe], acc.at[stripe])
        pltpu.sync_copy(edge_hbm.at[0, wid], idx_s)
        pltpu.sync_copy(edge_hbm.at[1, wid], idx_d)
        plsc.subcore_barrier()

        def gather(j, b):
            return pltpu.make_async_copy(
                table_hbm.at[idx_s.at[j]], rows.at[b], gsem.at[b])

        def scatter(j, b):
            return pltpu.make_async_copy(
                rows.at[b], acc.at[idx_d.at[j]], ssem.at[b])

        for b in range(NBUF):
            pltpu.async_copy(table_hbm.at[idx_s.at[b]], rows.at[b],
                             gsem.at[b])

        def body(t, carry):
            o = t * NBUF
            for b in range(NBUF):
                gather(o + b, b).wait()
                pltpu.async_copy(rows.at[b], acc.at[idx_d.at[o + b]],
                                 ssem.at[b], add=True)
            for b in range(NBUF):
                scatter(o + b, b).wait()
                pltpu.async_copy(table_hbm.at[idx_s.at[o + NBUF + b]],
                                 rows.at[b], gsem.at[b])
            return carry

        lax.fori_loop(0, NB // NBUF - 1, body, 0)

        o = NB - NBUF
        for b in range(NBUF):
            gather(o + b, b).wait()
            pltpu.async_copy(rows.at[b], acc.at[idx_d.at[o + b]],
                             ssem.at[b], add=True)
        for b in range(NBUF):
            scatter(o + b, b).wait()
        plsc.subcore_barrier()

        @pl.when(c == 0)
        def _():
            pltpu.sync_copy(acc.at[stripe], outA.at[stripe])

        @pl.when(c == 1)
        def _():
            pltpu.sync_copy(acc.at[stripe], outB.at[stripe])

    return scat_kernel(table, edges, zeros_init)


def _tc_mm1(x, W1):
    def body(x_ref, w_ref, out_ref):
        out_ref[...] = jnp.dot(x_ref[...], w_ref[...],
                               preferred_element_type=jnp.float32)

    return pl.pallas_call(
        body,
        grid=(1,),
        in_specs=[
            pl.BlockSpec((N, F), lambda j: (0, 0)),
            pl.BlockSpec((F, H), lambda j: (0, 0)),
        ],
        out_specs=pl.BlockSpec((N, H), lambda j: (0, 0)),
        out_shape=jax.ShapeDtypeStruct((N, H), jnp.float32),
    )(x, W1)


def _tc_scale(h1, degA, degB):
    def body(h_ref, da_ref, db_ref, out_ref, dinv_ref):
        deg = da_ref[:, 0:1] + db_ref[:, 0:1] + 1.0
        dinv = lax.rsqrt(deg)
        dinv_ref[...] = dinv
        out_ref[...] = h_ref[...] * dinv

    return pl.pallas_call(
        body,
        grid=(1,),
        in_specs=[
            pl.BlockSpec((N, H), lambda j: (0, 0)),
            pl.BlockSpec((N, DW), lambda j: (0, 0)),
            pl.BlockSpec((N, DW), lambda j: (0, 0)),
        ],
        out_specs=[
            pl.BlockSpec((N, H), lambda j: (0, 0)),
            pl.BlockSpec((N, 1), lambda j: (0, 0)),
        ],
        out_shape=[
            jax.ShapeDtypeStruct((N, H), jnp.float32),
            jax.ShapeDtypeStruct((N, 1), jnp.float32),
        ],
    )(h1, degA, degB)


def _tc_mid(Sa, Sb, hp, dinv, b, Wn):
    def body(sa_ref, sb_ref, hp_ref, dinv_ref, b_ref, w_ref, out_ref):
        dinv = dinv_ref[...]
        hcur = jnp.maximum(
            dinv * (sa_ref[...] + sb_ref[...] + hp_ref[...]) + b_ref[...], 0.0)
        out_ref[...] = jnp.dot(
            hcur, w_ref[...], preferred_element_type=jnp.float32) * dinv

    return pl.pallas_call(
        body,
        grid=(1,),
        in_specs=[
            pl.BlockSpec((N, H), lambda j: (0, 0)),
            pl.BlockSpec((N, H), lambda j: (0, 0)),
            pl.BlockSpec((N, H), lambda j: (0, 0)),
            pl.BlockSpec((N, 1), lambda j: (0, 0)),
            pl.BlockSpec((1, H), lambda j: (0, 0)),
            pl.BlockSpec((H, H), lambda j: (0, 0)),
        ],
        out_specs=pl.BlockSpec((N, H), lambda j: (0, 0)),
        out_shape=jax.ShapeDtypeStruct((N, H), jnp.float32),
    )(Sa, Sb, hp, dinv, b, Wn)


def _tc_final(Sa, Sb, hp, dinv, b):
    def body(sa_ref, sb_ref, hp_ref, dinv_ref, b_ref, out_ref):
        out_ref[...] = jnp.maximum(
            dinv_ref[...] * (sa_ref[...] + sb_ref[...] + hp_ref[...])
            + b_ref[...], 0.0)

    return pl.pallas_call(
        body,
        grid=(1,),
        in_specs=[
            pl.BlockSpec((N, H), lambda j: (0, 0)),
            pl.BlockSpec((N, H), lambda j: (0, 0)),
            pl.BlockSpec((N, H), lambda j: (0, 0)),
            pl.BlockSpec((N, 1), lambda j: (0, 0)),
            pl.BlockSpec((1, H), lambda j: (0, 0)),
        ],
        out_specs=pl.BlockSpec((N, H), lambda j: (0, 0)),
        out_shape=jax.ShapeDtypeStruct((N, H), jnp.float32),
    )(Sa, Sb, hp, dinv, b)


def _tc_head(z, Wc, bc):
    def body(zb_ref, zfull_ref, wc_ref, bc_ref, out_ref, sv_ref, acc_ref):
        j = pl.program_id(0)
        zb = zb_ref[...]
        s = lax.dot_general(zb, zfull_ref[...], (((1,), (1,)), ((), ())),
                            preferred_element_type=jnp.float32)
        out_ref[...] = 1.0 / (1.0 + jnp.exp(-s))

        @pl.when(j == 0)
        def _():
            acc_ref[...] = jnp.zeros_like(acc_ref)

        acc_ref[...] += jnp.sum(zb, axis=0, keepdims=True)
        sv_ref[...] = (jnp.dot(acc_ref[...], wc_ref[...],
                               preferred_element_type=jnp.float32)
                       * (1.0 / N)) + bc_ref[...]

    return pl.pallas_call(
        body,
        grid=(N // TM,),
        in_specs=[
            pl.BlockSpec((TM, H), lambda j: (j, 0)),
            pl.BlockSpec((N, H), lambda j: (0, 0)),
            pl.BlockSpec((H, 1), lambda j: (0, 0)),
            pl.BlockSpec((1, 1), lambda j: (0, 0)),
        ],
        out_specs=[
            pl.BlockSpec((TM, N), lambda j: (j, 0)),
            pl.BlockSpec((1, 1), lambda j: (0, 0)),
        ],
        out_shape=[
            jax.ShapeDtypeStruct((N, N), jnp.float32),
            jax.ShapeDtypeStruct((1, 1), jnp.float32),
        ],
        scratch_shapes=[pltpu.VMEM((1, H), jnp.float32)],
    )(z, z, Wc, bc)


def kernel(x, edge_index, W1, b1, W2, b2, W3, b3, Wc, bc):
    edges = edge_index.astype(jnp.int32).reshape(2, NTILES, NB, BLK)
    zeros_h = jnp.zeros((NACC, H), jnp.float32)
    zeros_d = jnp.zeros((NACC, DW), jnp.float32)
    ones_rows = jnp.ones((BLK, DW), jnp.float32)

    h1 = _tc_mm1(x, W1)
    degA, degB = _sc_degree(edges, zeros_d, ones_rows)

    h1p, dinv = _tc_scale(h1, degA, degB)
    S1a, S1b = _sc_scatter(h1p, edges, zeros_h)
    h2p = _tc_mid(S1a, S1b, h1p, dinv, b1.reshape(1, H), W2)
    S2a, S2b = _sc_scatter(h2p, edges, zeros_h)
    h3p = _tc_mid(S2a, S2b, h2p, dinv, b2.reshape(1, H), W3)
    S3a, S3b = _sc_scatter(h3p, edges, zeros_h)
    z = _tc_final(S3a, S3b, h3p, dinv, b3.reshape(1, H))

    scores, sv = _tc_head(z, Wc, bc.reshape(1, 1))
    return scores.reshape(-1), sv[0, 0]


# speedup vs baseline: 16.8853x; 1.0004x over previous
import functools

import jax
import jax.numpy as jnp
from jax import lax
from jax.experimental import pallas as pl
from jax.experimental.pallas import tpu as pltpu
from jax.experimental.pallas import tpu_sc as plsc

N = 10000
F = 128
H = 32
E = 320000

NTILES = 32
EPT = E // NTILES
BLK = 100
NB = EPT // BLK
NBUF = 10
NACC = 10112
STRIPE = NACC // 16
DW = 16

TM = 400


def _sc_mesh():
    return plsc.VectorSubcoreMesh(core_axis_name="c", subcore_axis_name="s")


def _sc_degree(edges, zeros_init, ones_rows):
    @functools.partial(
        pl.kernel,
        out_type=[jax.ShapeDtypeStruct((NACC, DW), jnp.float32),
                  jax.ShapeDtypeStruct((NACC, DW), jnp.float32)],
        mesh=_sc_mesh(),
        compiler_params=pltpu.CompilerParams(use_tc_tiling_on_sc=False),
        scratch_types=[
            pltpu.VMEM((NB, BLK), jnp.int32),
            pltpu.VMEM((BLK, DW), jnp.float32),
            pltpu.VMEM_SHARED((NACC, DW), jnp.float32),
        ],
    )
    def deg_kernel(edge_hbm, zeros_hbm, ones_hbm, outA, outB, idx_d, rows, acc):
        c = lax.axis_index("c")
        s = lax.axis_index("s")
        wid = c * 16 + s
        stripe = pl.ds(s * STRIPE, STRIPE)
        pltpu.sync_copy(zeros_hbm.at[stripe], acc.at[stripe])
        pltpu.sync_copy(edge_hbm.at[1, wid], idx_d)
        pltpu.sync_copy(ones_hbm, rows)
        plsc.subcore_barrier()

        def body(j, carry):
            pltpu.sync_copy(rows, acc.at[idx_d.at[j]], add=True)
            return carry

        lax.fori_loop(0, NB, body, 0)
        plsc.subcore_barrier()

        @pl.when(c == 0)
        def _():
            pltpu.sync_copy(acc.at[stripe], outA.at[stripe])

        @pl.when(c == 1)
        def _():
            pltpu.sync_copy(acc.at[stripe], outB.at[stripe])

    return deg_kernel(edges, zeros_init, ones_rows)


def _sc_scatter(table, edges, zeros_init):
    @functools.partial(
        pl.kernel,
        out_type=[jax.ShapeDtypeStruct((NACC, H), jnp.float32),
                  jax.ShapeDtypeStruct((NACC, H), jnp.float32)],
        mesh=_sc_mesh(),
        compiler_params=pltpu.CompilerParams(use_tc_tiling_on_sc=False),
        scratch_types=[
            pltpu.VMEM((NB, BLK), jnp.int32),
            pltpu.VMEM((NB, BLK), jnp.int32),
            pltpu.VMEM((NBUF, BLK, H), jnp.float32),
            pltpu.VMEM_SHARED((NACC, H), jnp.float32),
            pltpu.SemaphoreType.DMA((NBUF,)),
            pltpu.SemaphoreType.DMA((NBUF,)),
        ],
    )
    def scat_kernel(table_hbm, edge_hbm, zeros_hbm, outA, outB,
                    idx_s, idx_d, rows, acc, gsem, ssem):
        c = lax.axis_index("c")
        s = lax.axis_index("s")
        wid = c * 16 + s
        stripe = pl.ds(s * STRIPE, STRIPE)
        pltpu.sync_copy(zeros_hbm.at[stripe], acc.at[stripe])
        pltpu.sync_copy(edge_hbm.at[0, wid], idx_s)
        pltpu.sync_copy(edge_hbm.at[1, wid], idx_d)
        plsc.subcore_barrier()

        def gather(j, b):
            return pltpu.make_async_copy(
                table_hbm.at[idx_s.at[j]], rows.at[b], gsem.at[b])

        def scatter(j, b):
            return pltpu.make_async_copy(
                rows.at[b], acc.at[idx_d.at[j]], ssem.at[b])

        for b in range(NBUF):
            pltpu.async_copy(table_hbm.at[idx_s.at[b]], rows.at[b],
                             gsem.at[b])

        def body(t, carry):
            o = t * NBUF
            for b in range(NBUF):
                gather(o + b, b).wait()
                pltpu.async_copy(rows.at[b], acc.at[idx_d.at[o + b]],
                                 ssem.at[b], add=True)
            for b in range(NBUF):
                scatter(o + b, b).wait()
                pltpu.async_copy(table_hbm.at[idx_s.at[o + NBUF + b]],
                                 rows.at[b], gsem.at[b])
            return carry

        lax.fori_loop(0, NB // NBUF - 1, body, 0)

        o = NB - NBUF
        for b in range(NBUF):
            gather(o + b, b).wait()
            pltpu.async_copy(rows.at[b], acc.at[idx_d.at[o + b]],
                             ssem.at[b], add=True)
        for b in range(NBUF):
            scatter(o + b, b).wait()
        plsc.subcore_barrier()

        @pl.when(c == 0)
        def _():
            pltpu.sync_copy(acc.at[stripe], outA.at[stripe])

        @pl.when(c == 1)
        def _():
            pltpu.sync_copy(acc.at[stripe], outB.at[stripe])

    return scat_kernel(table, edges, zeros_init)


def _tc_prep(x, W1, degA, degB):
    def body(x_ref, w_ref, da_ref, db_ref, out_ref, dinv_ref):
        deg = da_ref[:, 0:1] + db_ref[:, 0:1] + 1.0
        dinv = lax.rsqrt(deg)
        dinv_ref[...] = dinv
        h = jnp.dot(x_ref[...], w_ref[...], preferred_element_type=jnp.float32)
        out_ref[...] = h * dinv

    return pl.pallas_call(
        body,
        grid=(1,),
        in_specs=[
            pl.BlockSpec((N, F), lambda j: (0, 0)),
            pl.BlockSpec((F, H), lambda j: (0, 0)),
            pl.BlockSpec((N, DW), lambda j: (0, 0)),
            pl.BlockSpec((N, DW), lambda j: (0, 0)),
        ],
        out_specs=[
            pl.BlockSpec((N, H), lambda j: (0, 0)),
            pl.BlockSpec((N, 1), lambda j: (0, 0)),
        ],
        out_shape=[
            jax.ShapeDtypeStruct((N, H), jnp.float32),
            jax.ShapeDtypeStruct((N, 1), jnp.float32),
        ],
    )(x, W1, degA, degB)


def _tc_mid(Sa, Sb, hp, dinv, b, Wn):
    def body(sa_ref, sb_ref, hp_ref, dinv_ref, b_ref, w_ref, out_ref):
        dinv = dinv_ref[...]
        hcur = jnp.maximum(
            dinv * (sa_ref[...] + sb_ref[...] + hp_ref[...]) + b_ref[...], 0.0)
        out_ref[...] = jnp.dot(
            hcur, w_ref[...], preferred_element_type=jnp.float32) * dinv

    return pl.pallas_call(
        body,
        grid=(1,),
        in_specs=[
            pl.BlockSpec((N, H), lambda j: (0, 0)),
            pl.BlockSpec((N, H), lambda j: (0, 0)),
            pl.BlockSpec((N, H), lambda j: (0, 0)),
            pl.BlockSpec((N, 1), lambda j: (0, 0)),
            pl.BlockSpec((1, H), lambda j: (0, 0)),
            pl.BlockSpec((H, H), lambda j: (0, 0)),
        ],
        out_specs=pl.BlockSpec((N, H), lambda j: (0, 0)),
        out_shape=jax.ShapeDtypeStruct((N, H), jnp.float32),
    )(Sa, Sb, hp, dinv, b, Wn)


def _tc_final(Sa, Sb, hp, dinv, b):
    def body(sa_ref, sb_ref, hp_ref, dinv_ref, b_ref, out_ref):
        out_ref[...] = jnp.maximum(
            dinv_ref[...] * (sa_ref[...] + sb_ref[...] + hp_ref[...])
            + b_ref[...], 0.0)

    return pl.pallas_call(
        body,
        grid=(1,),
        in_specs=[
            pl.BlockSpec((N, H), lambda j: (0, 0)),
            pl.BlockSpec((N, H), lambda j: (0, 0)),
            pl.BlockSpec((N, H), lambda j: (0, 0)),
            pl.BlockSpec((N, 1), lambda j: (0, 0)),
            pl.BlockSpec((1, H), lambda j: (0, 0)),
        ],
        out_specs=pl.BlockSpec((N, H), lambda j: (0, 0)),
        out_shape=jax.ShapeDtypeStruct((N, H), jnp.float32),
    )(Sa, Sb, hp, dinv, b)


def _tc_head(z, Wc, bc):
    def body(zb_ref, zfull_ref, wc_ref, bc_ref, out_ref, sv_ref, acc_ref):
        j = pl.program_id(0)
        zb = zb_ref[...]
        s = lax.dot_general(zb, zfull_ref[...], (((1,), (1,)), ((), ())),
                            preferred_element_type=jnp.float32)
        out_ref[...] = 1.0 / (1.0 + jnp.exp(-s))

        @pl.when(j == 0)
        def _():
            acc_ref[...] = jnp.zeros_like(acc_ref)

        acc_ref[...] += jnp.sum(zb, axis=0, keepdims=True)
        sv_ref[...] = (jnp.dot(acc_ref[...], wc_ref[...],
                               preferred_element_type=jnp.float32)
                       * (1.0 / N)) + bc_ref[...]

    return pl.pallas_call(
        body,
        grid=(N // TM,),
        in_specs=[
            pl.BlockSpec((TM, H), lambda j: (j, 0)),
            pl.BlockSpec((N, H), lambda j: (0, 0)),
            pl.BlockSpec((H, 1), lambda j: (0, 0)),
            pl.BlockSpec((1, 1), lambda j: (0, 0)),
        ],
        out_specs=[
            pl.BlockSpec((TM, N), lambda j: (j, 0)),
            pl.BlockSpec((1, 1), lambda j: (0, 0)),
        ],
        out_shape=[
            jax.ShapeDtypeStruct((N, N), jnp.float32),
            jax.ShapeDtypeStruct((1, 1), jnp.float32),
        ],
        scratch_shapes=[pltpu.VMEM((1, H), jnp.float32)],
    )(z, z, Wc, bc)


def kernel(x, edge_index, W1, b1, W2, b2, W3, b3, Wc, bc):
    edges = edge_index.astype(jnp.int32).reshape(2, NTILES, NB, BLK)
    zeros_h = jnp.zeros((NACC, H), jnp.float32)
    zeros_d = jnp.zeros((NACC, DW), jnp.float32)
    ones_rows = jnp.ones((BLK, DW), jnp.float32)

    degA, degB = _sc_degree(edges, zeros_d, ones_rows)

    h1p, dinv = _tc_prep(x, W1, degA, degB)
    S1a, S1b = _sc_scatter(h1p, edges, zeros_h)
    h2p = _tc_mid(S1a, S1b, h1p, dinv, b1.reshape(1, H), W2)
    S2a, S2b = _sc_scatter(h2p, edges, zeros_h)
    h3p = _tc_mid(S2a, S2b, h2p, dinv, b2.reshape(1, H), W3)
    S3a, S3b = _sc_scatter(h3p, edges, zeros_h)
    z = _tc_final(S3a, S3b, h3p, dinv, b3.reshape(1, H))

    scores, sv = _tc_head(z, Wc, bc.reshape(1, 1))
    return scores.reshape(-1), sv[0, 0]
